# baseline (speedup 1.0000x reference)
import functools

import jax
import jax.numpy as jnp
from jax import lax
from jax.experimental import pallas as pl
from jax.experimental.pallas import tpu as pltpu
from jax.experimental.pallas import tpu_sc as plsc

NUM_TILES = 32
CH = 128


def _make_sc_edge_kernel(BN, EMB, EPAD, NPT, NODEPAD, RPPAD):
    mesh = plsc.VectorSubcoreMesh(core_axis_name="c", subcore_axis_name="s")
    nk = EMB // 16

    @functools.partial(
        pl.kernel,
        out_type=jax.ShapeDtypeStruct((NODEPAD, EMB), jnp.float32),
        mesh=mesh,
        scratch_types=[
            pltpu.VMEM((336,), jnp.int32),
            pltpu.VMEM((3, CH + 16), jnp.int32),
            pltpu.VMEM((CH, EMB), jnp.float32),
            pltpu.VMEM((CH, EMB), jnp.float32),
            pltpu.VMEM((NPT, EMB), jnp.float32),
        ],
        compiler_params=pltpu.CompilerParams(use_tc_tiling_on_sc=False),
    )
    def sc_edge(h_hbm, idx_hbm, rowptr_hbm, ea_hbm, out_hbm,
                rp_v, idx_v, hbuf, eabuf, outbuf):
        core = lax.axis_index("c")
        sub = lax.axis_index("s")
        tid = core * 16 + sub
        n0 = tid * NPT
        a0 = (n0 // 8) * 8
        pltpu.sync_copy(rowptr_hbm.at[pl.ds(a0, 336)], rp_v)
        off = n0 - a0
        e0 = rp_v[pl.ds(off, 16)][0]
        e1 = rp_v[pl.ds(off + NPT, 16)][0]

        @pl.loop(0, NPT)
        def _(i):
            for k in range(nk):
                outbuf[i, pl.ds(k * 16, 16)] = jnp.zeros((16,), jnp.float32)

        e0a = (e0 // 8) * 8
        nch = (e1 - e0a + (CH - 1)) // CH

        @pl.loop(0, nch)
        def _(c):
            eb = e0a + c * CH
            pltpu.sync_copy(idx_hbm.at[:, pl.ds(eb, CH)],
                            idx_v.at[:, pl.ds(0, CH)])
            pltpu.sync_copy(h_hbm.at[idx_v.at[0, pl.ds(0, CH)]], hbuf)
            pltpu.sync_copy(ea_hbm.at[idx_v.at[1, pl.ds(0, CH)]], eabuf)
            jlo = jnp.maximum(e0 - eb, 0)
            jhi = jnp.minimum(e1 - eb, CH)

            @pl.loop(jlo, jhi)
            def _(j):
                rloc = idx_v[2, pl.ds(j, 16)][0] - n0
                for k in range(nk):
                    sl = pl.ds(k * 16, 16)
                    m = jnp.maximum(hbuf[j, sl] + eabuf[j, sl], 0.0)
                    outbuf[rloc, sl] = outbuf[rloc, sl] + m

        pltpu.sync_copy(outbuf, out_hbm.at[pl.ds(n0, NPT)])

    return sc_edge


def _tc_mlp(h, agg, W1, b1, W2, b2, last):
    BN, EMB = h.shape
    H2 = W1.shape[1]
    RB = 1000

    def body(h_ref, a_ref, w1_ref, b1_ref, w2_ref, b2_ref, o_ref):
        z = h_ref[...] + a_ref[...]
        z1 = jnp.dot(z, w1_ref[...], preferred_element_type=jnp.float32) + b1_ref[...]
        z1 = jnp.maximum(z1, 0.0)
        z2 = jnp.dot(z1, w2_ref[...], preferred_element_type=jnp.float32) + b2_ref[...]
        o_ref[...] = z2 if last else jnp.maximum(z2, 0.0)

    return pl.pallas_call(
        body,
        grid=(BN // RB,),
        in_specs=[
            pl.BlockSpec((RB, EMB), lambda i: (i, 0)),
            pl.BlockSpec((RB, EMB), lambda i: (i, 0)),
            pl.BlockSpec((EMB, H2), lambda i: (0, 0)),
            pl.BlockSpec((1, H2), lambda i: (0, 0)),
            pl.BlockSpec((H2, EMB), lambda i: (0, 0)),
            pl.BlockSpec((1, EMB), lambda i: (0, 0)),
        ],
        out_specs=pl.BlockSpec((RB, EMB), lambda i: (i, 0)),
        out_shape=jax.ShapeDtypeStruct((BN, EMB), jnp.float32),
    )(h, agg, W1, b1, W2, b2)


def _tc_pool(h3, W_cp, b_cp, W_out, b_out):
    B, N, EMB = h3.shape
    RANK = W_cp.shape[1]
    TASKS = W_out.shape[1]
    PAD = 1
    while PAD < N:
        PAD *= 2

    def body(h_ref, wcp_ref, bcp_ref, wout_ref, bout_ref, o_ref):
        t = jnp.tanh(jnp.dot(h_ref[0], wcp_ref[...],
                             preferred_element_type=jnp.float32) + bcp_ref[...])
        acc = jnp.concatenate(
            [t, jnp.ones((PAD - N, RANK), jnp.float32)], axis=0)
        n = PAD
        while n > 1:
            n //= 2
            acc = acc[:n] * acc[n:2 * n]
        row = jnp.dot(acc, wout_ref[...],
                      preferred_element_type=jnp.float32) + bout_ref[...]
        o_ref[pl.ds(pl.program_id(0), 1), :] = row

    return pl.pallas_call(
        body,
        grid=(B,),
        in_specs=[
            pl.BlockSpec((1, N, EMB), lambda i: (i, 0, 0)),
            pl.BlockSpec((EMB, RANK), lambda i: (0, 0)),
            pl.BlockSpec((1, RANK), lambda i: (0, 0)),
            pl.BlockSpec((RANK, TASKS), lambda i: (0, 0)),
            pl.BlockSpec((1, TASKS), lambda i: (0, 0)),
        ],
        out_specs=pl.BlockSpec((B, TASKS), lambda i: (0, 0)),
        out_shape=jax.ShapeDtypeStruct((B, TASKS), jnp.float32),
    )(h3, W_cp, b_cp, W_out, b_out)


@jax.jit
def kernel(x, e_idx, e_attr, params):
    B, N, EMB = x.shape
    E = e_idx.shape[2]
    TOTAL_E = B * E
    BN = B * N
    NPT = -(-BN // NUM_TILES)
    NODEPAD = NPT * NUM_TILES
    EPAD = TOTAL_E + 3 * CH
    RPPAD = ((NODEPAD + 336) // 8) * 8 + 8

    g_offs = (jnp.arange(B, dtype=jnp.int32) * N)[:, None]
    dstg = (e_idx[:, 1, :].astype(jnp.int32) + g_offs).reshape(-1)
    srcg = (e_idx[:, 0, :].astype(jnp.int32) + g_offs).reshape(-1)
    perm = jnp.argsort(dstg, stable=True).astype(jnp.int32)
    dst_s = jnp.take(dstg, perm)
    src_s = jnp.take(srcg, perm)
    rowptr = jnp.searchsorted(
        dst_s, jnp.arange(NODEPAD + 1, dtype=jnp.int32)).astype(jnp.int32)
    zpad = jnp.zeros((EPAD - TOTAL_E,), jnp.int32)
    idx3 = jnp.stack([
        jnp.concatenate([src_s, zpad]),
        jnp.concatenate([perm, zpad]),
        jnp.concatenate([dst_s, zpad]),
    ])
    rowptr_p = jnp.concatenate(
        [rowptr, jnp.full((RPPAD - NODEPAD - 1,), TOTAL_E, jnp.int32)])

    ea_flat = e_attr.reshape(TOTAL_E, EMB)
    h = x.reshape(BN, EMB)

    sc_edge = _make_sc_edge_kernel(BN, EMB, EPAD, NPT, NODEPAD, RPPAD)

    n_layers = len(params["layers"])
    for l, (W1, b1, W2, b2) in enumerate(params["layers"]):
        agg = sc_edge(h, idx3, rowptr_p, ea_flat)[:BN]
        h = _tc_mlp(h, agg, W1, b1[None], W2, b2[None], l == n_layers - 1)

    return _tc_pool(h.reshape(B, N, EMB), params["W_cp"], params["b_cp"][None],
                    params["W_out"], params["b_out"][None])


# speedup vs baseline: 1.4102x; 1.4102x over previous
import functools

import jax
import jax.numpy as jnp
from jax import lax
from jax.experimental import pallas as pl
from jax.experimental.pallas import tpu as pltpu
from jax.experimental.pallas import tpu_sc as plsc

NUM_TILES = 32
CH = 128


def _make_sc_edge_kernel(BN, EMB, EPAD, NPT, NODEPAD, RPPAD):
    mesh = plsc.VectorSubcoreMesh(core_axis_name="c", subcore_axis_name="s")
    nk = EMB // 16

    @functools.partial(
        pl.kernel,
        out_type=jax.ShapeDtypeStruct((NODEPAD, EMB), jnp.float32),
        mesh=mesh,
        scratch_types=[
            pltpu.VMEM((336,), jnp.int32),
            pltpu.VMEM((2, 3, CH), jnp.int32),
            pltpu.VMEM((2, CH, EMB), jnp.float32),
            pltpu.VMEM((2, CH, EMB), jnp.float32),
            pltpu.VMEM((2, CH + 16), jnp.int32),
            pltpu.VMEM((NPT, EMB), jnp.float32),
            pltpu.SemaphoreType.DMA,
            pltpu.SemaphoreType.DMA,
            pltpu.SemaphoreType.DMA,
            pltpu.SemaphoreType.DMA,
        ],
        compiler_params=pltpu.CompilerParams(use_tc_tiling_on_sc=False),
    )
    def sc_edge(h_hbm, idx_hbm, rowptr_hbm, ea_hbm, out_hbm,
                rp_v, idx_v, hbuf, eabuf, dstb, outbuf,
                sem_g0, sem_g1, sem_i0, sem_i1):
        core = lax.axis_index("c")
        sub = lax.axis_index("s")
        tid = core * 16 + sub
        n0 = tid * NPT
        a0 = (n0 // 8) * 8
        pltpu.sync_copy(rowptr_hbm.at[pl.ds(a0, 336)], rp_v)
        off = n0 - a0
        e0 = rp_v[pl.ds(off, 16)][0]
        e1 = rp_v[pl.ds(off + NPT, 16)][0]

        @pl.loop(0, NPT)
        def _(i):
            for k in range(nk):
                outbuf[i, pl.ds(k * 16, 16)] = jnp.zeros((16,), jnp.float32)

        e0a = (e0 // 8) * 8
        nch = (e1 - e0a + (CH - 1)) // CH
        nch2 = (nch + 1) // 2
        sem_g = (sem_g0, sem_g1)
        sem_i = (sem_i0, sem_i1)

        def issue_idx(c, s):
            eb = e0a + c * CH
            pltpu.async_copy(idx_hbm.at[:, pl.ds(eb, CH)], idx_v.at[s],
                             sem_i[s])

        def wait_idx(s):
            pltpu.make_async_copy(idx_hbm.at[:, pl.ds(0, CH)], idx_v.at[s],
                                  sem_i[s]).wait()

        def issue_gath(s):
            pltpu.async_copy(h_hbm.at[idx_v.at[s, 0]], hbuf.at[s], sem_g[s])
            pltpu.async_copy(ea_hbm.at[idx_v.at[s, 1]], eabuf.at[s], sem_g[s])

        def wait_gath(s):
            pltpu.make_async_copy(h_hbm.at[idx_v.at[s, 0]], hbuf.at[s],
                                  sem_g[s]).wait()
            pltpu.make_async_copy(ea_hbm.at[idx_v.at[s, 1]], eabuf.at[s],
                                  sem_g[s]).wait()

        def copy_dst(s):
            for k in range(CH // 16):
                dstb[s, pl.ds(k * 16, 16)] = idx_v[s, 2, pl.ds(k * 16, 16)]

        def accum(c, s):
            eb = e0a + c * CH
            jlo = jnp.maximum(e0 - eb, 0)
            jhi = jnp.minimum(e1 - eb, CH)

            @pl.loop(jlo, jhi)
            def _(j):
                rloc = dstb[s, pl.ds(j, 16)][0] - n0
                for k in range(nk):
                    sl = pl.ds(k * 16, 16)
                    m = jnp.maximum(hbuf[s, j, sl] + eabuf[s, j, sl], 0.0)
                    plsc.addupdate(outbuf.at[rloc, sl], m)

        pltpu.sync_copy(idx_hbm.at[:, pl.ds(e0a, CH)], idx_v.at[0])
        issue_gath(0)
        issue_idx(1, 1)

        @pl.loop(0, nch2)
        def _(c2):
            c = 2 * c2
            wait_gath(0)
            copy_dst(0)
            wait_idx(1)
            issue_gath(1)
            issue_idx(c + 2, 0)
            accum(c, 0)
            wait_gath(1)
            copy_dst(1)
            wait_idx(0)
            issue_gath(0)
            issue_idx(c + 3, 1)
            accum(c + 1, 1)

        wait_gath(0)
        wait_idx(1)

        pltpu.sync_copy(outbuf, out_hbm.at[pl.ds(n0, NPT)])

    return sc_edge


def _tc_mlp(h, agg, W1, b1, W2, b2, last):
    BN, EMB = h.shape
    H2 = W1.shape[1]
    RB = 1000

    def body(h_ref, a_ref, w1_ref, b1_ref, w2_ref, b2_ref, o_ref):
        z = h_ref[...] + a_ref[...]
        z1 = jnp.dot(z, w1_ref[...], preferred_element_type=jnp.float32) + b1_ref[...]
        z1 = jnp.maximum(z1, 0.0)
        z2 = jnp.dot(z1, w2_ref[...], preferred_element_type=jnp.float32) + b2_ref[...]
        o_ref[...] = z2 if last else jnp.maximum(z2, 0.0)

    return pl.pallas_call(
        body,
        grid=(BN // RB,),
        in_specs=[
            pl.BlockSpec((RB, EMB), lambda i: (i, 0)),
            pl.BlockSpec((RB, EMB), lambda i: (i, 0)),
            pl.BlockSpec((EMB, H2), lambda i: (0, 0)),
            pl.BlockSpec((1, H2), lambda i: (0, 0)),
            pl.BlockSpec((H2, EMB), lambda i: (0, 0)),
            pl.BlockSpec((1, EMB), lambda i: (0, 0)),
        ],
        out_specs=pl.BlockSpec((RB, EMB), lambda i: (i, 0)),
        out_shape=jax.ShapeDtypeStruct((BN, EMB), jnp.float32),
    )(h, agg, W1, b1, W2, b2)


def _tc_pool(h3, W_cp, b_cp, W_out, b_out):
    B, N, EMB = h3.shape
    RANK = W_cp.shape[1]
    TASKS = W_out.shape[1]
    PAD = 1
    while PAD < N:
        PAD *= 2

    def body(h_ref, wcp_ref, bcp_ref, wout_ref, bout_ref, o_ref):
        t = jnp.tanh(jnp.dot(h_ref[0], wcp_ref[...],
                             preferred_element_type=jnp.float32) + bcp_ref[...])
        acc = jnp.concatenate(
            [t, jnp.ones((PAD - N, RANK), jnp.float32)], axis=0)
        n = PAD
        while n > 1:
            n //= 2
            acc = acc[:n] * acc[n:2 * n]
        row = jnp.dot(acc, wout_ref[...],
                      preferred_element_type=jnp.float32) + bout_ref[...]
        o_ref[pl.ds(pl.program_id(0), 1), :] = row

    return pl.pallas_call(
        body,
        grid=(B,),
        in_specs=[
            pl.BlockSpec((1, N, EMB), lambda i: (i, 0, 0)),
            pl.BlockSpec((EMB, RANK), lambda i: (0, 0)),
            pl.BlockSpec((1, RANK), lambda i: (0, 0)),
            pl.BlockSpec((RANK, TASKS), lambda i: (0, 0)),
            pl.BlockSpec((1, TASKS), lambda i: (0, 0)),
        ],
        out_specs=pl.BlockSpec((B, TASKS), lambda i: (0, 0)),
        out_shape=jax.ShapeDtypeStruct((B, TASKS), jnp.float32),
    )(h3, W_cp, b_cp, W_out, b_out)


@jax.jit
def kernel(x, e_idx, e_attr, params):
    B, N, EMB = x.shape
    E = e_idx.shape[2]
    TOTAL_E = B * E
    BN = B * N
    NPT = -(-BN // NUM_TILES)
    NODEPAD = NPT * NUM_TILES
    EPAD = TOTAL_E + 3 * CH
    RPPAD = ((NODEPAD + 336) // 8) * 8 + 8

    g_offs = (jnp.arange(B, dtype=jnp.int32) * N)[:, None]
    dstg = (e_idx[:, 1, :].astype(jnp.int32) + g_offs).reshape(-1)
    srcg = (e_idx[:, 0, :].astype(jnp.int32) + g_offs).reshape(-1)
    perm = jnp.argsort(dstg, stable=True).astype(jnp.int32)
    dst_s = jnp.take(dstg, perm)
    src_s = jnp.take(srcg, perm)
    rowptr = jnp.searchsorted(
        dst_s, jnp.arange(NODEPAD + 1, dtype=jnp.int32)).astype(jnp.int32)
    zpad = jnp.zeros((EPAD - TOTAL_E,), jnp.int32)
    idx3 = jnp.stack([
        jnp.concatenate([src_s, zpad]),
        jnp.concatenate([perm, zpad]),
        jnp.concatenate([dst_s, zpad]),
    ])
    rowptr_p = jnp.concatenate(
        [rowptr, jnp.full((RPPAD - NODEPAD - 1,), TOTAL_E, jnp.int32)])

    ea_flat = e_attr.reshape(TOTAL_E, EMB)
    h = x.reshape(BN, EMB)

    sc_edge = _make_sc_edge_kernel(BN, EMB, EPAD, NPT, NODEPAD, RPPAD)

    n_layers = len(params["layers"])
    for l, (W1, b1, W2, b2) in enumerate(params["layers"]):
        agg = sc_edge(h, idx3, rowptr_p, ea_flat)[:BN]
        h = _tc_mlp(h, agg, W1, b1[None], W2, b2[None], l == n_layers - 1)

    return _tc_pool(h.reshape(B, N, EMB), params["W_cp"], params["b_cp"][None],
                    params["W_out"], params["b_out"][None])


# speedup vs baseline: 1.4213x; 1.0078x over previous
import functools

import jax
import jax.numpy as jnp
from jax import lax
from jax.experimental import pallas as pl
from jax.experimental.pallas import tpu as pltpu
from jax.experimental.pallas import tpu_sc as plsc

NUM_TILES = 32
CH = 128


def _make_sc_edge_kernel(BN, EMB, EPAD, NPT, NODEPAD, RPPAD):
    mesh = plsc.VectorSubcoreMesh(core_axis_name="c", subcore_axis_name="s")
    nk = EMB // 16

    @functools.partial(
        pl.kernel,
        out_type=jax.ShapeDtypeStruct((NODEPAD, EMB), jnp.float32),
        mesh=mesh,
        scratch_types=[
            pltpu.VMEM((336,), jnp.int32),
            pltpu.VMEM((2, 3, CH), jnp.int32),
            pltpu.VMEM((2, CH, EMB), jnp.float32),
            pltpu.VMEM((2, CH, EMB), jnp.float32),
            pltpu.VMEM((2, CH + 16), jnp.int32),
            pltpu.VMEM((NPT, EMB), jnp.float32),
            pltpu.SemaphoreType.DMA,
            pltpu.SemaphoreType.DMA,
            pltpu.SemaphoreType.DMA,
            pltpu.SemaphoreType.DMA,
        ],
        compiler_params=pltpu.CompilerParams(use_tc_tiling_on_sc=False),
    )
    def sc_edge(h_hbm, idx_hbm, rowptr_hbm, ea_hbm, out_hbm,
                rp_v, idx_v, hbuf, eabuf, dstb, outbuf,
                sem_g0, sem_g1, sem_i0, sem_i1):
        core = lax.axis_index("c")
        sub = lax.axis_index("s")
        tid = core * 16 + sub
        n0 = tid * NPT
        a0 = (n0 // 8) * 8
        pltpu.sync_copy(rowptr_hbm.at[pl.ds(a0, 336)], rp_v)
        off = n0 - a0
        e0 = rp_v[pl.ds(off, 16)][0]
        e1 = rp_v[pl.ds(off + NPT, 16)][0]

        @pl.loop(0, NPT)
        def _(i):
            for k in range(nk):
                outbuf[i, pl.ds(k * 16, 16)] = jnp.zeros((16,), jnp.float32)

        e0a = (e0 // 8) * 8
        nch = (e1 - e0a + (CH - 1)) // CH
        nch2 = (nch + 1) // 2
        sem_g = (sem_g0, sem_g1)
        sem_i = (sem_i0, sem_i1)

        def issue_idx(c, s):
            eb = e0a + c * CH
            pltpu.async_copy(idx_hbm.at[:, pl.ds(eb, CH)], idx_v.at[s],
                             sem_i[s])

        def wait_idx(s):
            pltpu.make_async_copy(idx_hbm.at[:, pl.ds(0, CH)], idx_v.at[s],
                                  sem_i[s]).wait()

        def issue_gath(s):
            pltpu.async_copy(h_hbm.at[idx_v.at[s, 0]], hbuf.at[s], sem_g[s])
            pltpu.async_copy(ea_hbm.at[idx_v.at[s, 1]], eabuf.at[s], sem_g[s])

        def wait_gath(s):
            pltpu.make_async_copy(h_hbm.at[idx_v.at[s, 0]], hbuf.at[s],
                                  sem_g[s]).wait()
            pltpu.make_async_copy(ea_hbm.at[idx_v.at[s, 1]], eabuf.at[s],
                                  sem_g[s]).wait()

        def copy_dst(s):
            n0v = jnp.full((16,), n0, jnp.int32)
            for k in range(CH // 16):
                dstb[s, pl.ds(k * 16, 16)] = (
                    idx_v[s, 2, pl.ds(k * 16, 16)] - n0v)

        def accum(c, s):
            eb = e0a + c * CH
            jlo = jnp.maximum(e0 - eb, 0)
            jhi = jnp.minimum(e1 - eb, CH)

            def edge_body(j):
                rloc = dstb[s, pl.ds(j, 16)][0]
                for k in range(nk):
                    sl = pl.ds(k * 16, 16)
                    m = jnp.maximum(hbuf[s, j, sl] + eabuf[s, j, sl], 0.0)
                    plsc.addupdate(outbuf.at[rloc, sl], m)

            interior = jnp.logical_and(jlo == 0, jhi == CH)

            @pl.when(interior)
            def _():
                @pl.loop(0, CH, unroll=4)
                def _(j):
                    edge_body(j)

            @pl.when(jnp.logical_not(interior))
            def _():
                @pl.loop(jlo, jhi)
                def _(j):
                    edge_body(j)

        pltpu.sync_copy(idx_hbm.at[:, pl.ds(e0a, CH)], idx_v.at[0])
        issue_gath(0)
        issue_idx(1, 1)

        @pl.loop(0, nch2)
        def _(c2):
            c = 2 * c2
            wait_gath(0)
            copy_dst(0)
            wait_idx(1)
            issue_gath(1)
            issue_idx(c + 2, 0)
            accum(c, 0)
            wait_gath(1)
            copy_dst(1)
            wait_idx(0)
            issue_gath(0)
            issue_idx(c + 3, 1)
            accum(c + 1, 1)

        wait_gath(0)
        wait_idx(1)

        pltpu.sync_copy(outbuf, out_hbm.at[pl.ds(n0, NPT)])

    return sc_edge


def _tc_mlp(h, agg, W1, b1, W2, b2, last):
    BN, EMB = h.shape
    H2 = W1.shape[1]
    RB = 1000

    def body(h_ref, a_ref, w1_ref, b1_ref, w2_ref, b2_ref, o_ref):
        z = h_ref[...] + a_ref[...]
        z1 = jnp.dot(z, w1_ref[...], preferred_element_type=jnp.float32) + b1_ref[...]
        z1 = jnp.maximum(z1, 0.0)
        z2 = jnp.dot(z1, w2_ref[...], preferred_element_type=jnp.float32) + b2_ref[...]
        o_ref[...] = z2 if last else jnp.maximum(z2, 0.0)

    return pl.pallas_call(
        body,
        grid=(BN // RB,),
        in_specs=[
            pl.BlockSpec((RB, EMB), lambda i: (i, 0)),
            pl.BlockSpec((RB, EMB), lambda i: (i, 0)),
            pl.BlockSpec((EMB, H2), lambda i: (0, 0)),
            pl.BlockSpec((1, H2), lambda i: (0, 0)),
            pl.BlockSpec((H2, EMB), lambda i: (0, 0)),
            pl.BlockSpec((1, EMB), lambda i: (0, 0)),
        ],
        out_specs=pl.BlockSpec((RB, EMB), lambda i: (i, 0)),
        out_shape=jax.ShapeDtypeStruct((BN, EMB), jnp.float32),
    )(h, agg, W1, b1, W2, b2)


def _tc_pool(h3, W_cp, b_cp, W_out, b_out):
    B, N, EMB = h3.shape
    RANK = W_cp.shape[1]
    TASKS = W_out.shape[1]
    PAD = 1
    while PAD < N:
        PAD *= 2

    def body(h_ref, wcp_ref, bcp_ref, wout_ref, bout_ref, o_ref):
        t = jnp.tanh(jnp.dot(h_ref[0], wcp_ref[...],
                             preferred_element_type=jnp.float32) + bcp_ref[...])
        acc = jnp.concatenate(
            [t, jnp.ones((PAD - N, RANK), jnp.float32)], axis=0)
        n = PAD
        while n > 1:
            n //= 2
            acc = acc[:n] * acc[n:2 * n]
        row = jnp.dot(acc, wout_ref[...],
                      preferred_element_type=jnp.float32) + bout_ref[...]
        o_ref[pl.ds(pl.program_id(0), 1), :] = row

    return pl.pallas_call(
        body,
        grid=(B,),
        in_specs=[
            pl.BlockSpec((1, N, EMB), lambda i: (i, 0, 0)),
            pl.BlockSpec((EMB, RANK), lambda i: (0, 0)),
            pl.BlockSpec((1, RANK), lambda i: (0, 0)),
            pl.BlockSpec((RANK, TASKS), lambda i: (0, 0)),
            pl.BlockSpec((1, TASKS), lambda i: (0, 0)),
        ],
        out_specs=pl.BlockSpec((B, TASKS), lambda i: (0, 0)),
        out_shape=jax.ShapeDtypeStruct((B, TASKS), jnp.float32),
    )(h3, W_cp, b_cp, W_out, b_out)


@jax.jit
def kernel(x, e_idx, e_attr, params):
    B, N, EMB = x.shape
    E = e_idx.shape[2]
    TOTAL_E = B * E
    BN = B * N
    NPT = -(-BN // NUM_TILES)
    NODEPAD = NPT * NUM_TILES
    EPAD = TOTAL_E + 3 * CH
    RPPAD = ((NODEPAD + 336) // 8) * 8 + 8

    g_offs = (jnp.arange(B, dtype=jnp.int32) * N)[:, None]
    dstg = (e_idx[:, 1, :].astype(jnp.int32) + g_offs).reshape(-1)
    srcg = (e_idx[:, 0, :].astype(jnp.int32) + g_offs).reshape(-1)
    perm = jnp.argsort(dstg, stable=True).astype(jnp.int32)
    dst_s = jnp.take(dstg, perm)
    src_s = jnp.take(srcg, perm)
    rowptr = jnp.searchsorted(
        dst_s, jnp.arange(NODEPAD + 1, dtype=jnp.int32)).astype(jnp.int32)
    zpad = jnp.zeros((EPAD - TOTAL_E,), jnp.int32)
    idx3 = jnp.stack([
        jnp.concatenate([src_s, zpad]),
        jnp.concatenate([perm, zpad]),
        jnp.concatenate([dst_s, zpad]),
    ])
    rowptr_p = jnp.concatenate(
        [rowptr, jnp.full((RPPAD - NODEPAD - 1,), TOTAL_E, jnp.int32)])

    ea_flat = e_attr.reshape(TOTAL_E, EMB)
    h = x.reshape(BN, EMB)

    sc_edge = _make_sc_edge_kernel(BN, EMB, EPAD, NPT, NODEPAD, RPPAD)

    n_layers = len(params["layers"])
    for l, (W1, b1, W2, b2) in enumerate(params["layers"]):
        agg = sc_edge(h, idx3, rowptr_p, ea_flat)[:BN]
        h = _tc_mlp(h, agg, W1, b1[None], W2, b2[None], l == n_layers - 1)

    return _tc_pool(h.reshape(B, N, EMB), params["W_cp"], params["b_cp"][None],
                    params["W_out"], params["b_out"][None])
